# baseline (speedup 1.0000x reference)
import functools

import jax
import jax.numpy as jnp
from jax import lax
from jax.experimental import pallas as pl
from jax.experimental.pallas import tpu as pltpu
from jax.experimental.pallas import tpu_sc as plsc

GAMMA = 1.0
CHUNK = 128
D = 64
ENT = 1000000
NPAIR = ENT // 2
WIN = 768
NFULL = ENT // WIN
TAIL_E = NFULL * WIN
BIN_CAP = 3264
SL_CAP = 160
BIG = 0x7FFFFFFF


def _vec_sqrt(x):
    i = lax.bitcast_convert_type(x, jnp.int32)
    y = lax.bitcast_convert_type(
        jnp.int32(0x5F3759DF) - lax.shift_right_arithmetic(i, 1), jnp.float32
    )
    for _ in range(3):
        y = y * (1.5 - 0.5 * x * y * y)
    return x * y


def _sc_extract(ent_t, ent_t_tail, pair_idx_2d, num_cores, num_subcores):
    mesh = plsc.VectorSubcoreMesh(core_axis_name="c", subcore_axis_name="s")
    nw = num_cores * num_subcores
    n_rows = pair_idx_2d[0].shape[0]
    n_win = NFULL + 1
    base_cnt, extra = divmod(n_win, nw)

    @functools.partial(
        pl.kernel,
        out_type=jax.ShapeDtypeStruct((NPAIR, 2 * D), jnp.float32),
        mesh=mesh,
        scratch_types=[
            pltpu.VMEM((32, 128), jnp.int32),
            pltpu.VMEM((BIN_CAP,), jnp.int32),
            pltpu.VMEM((SL_CAP,), jnp.int32),
            pltpu.VMEM((D, WIN), jnp.float32),
            pltpu.VMEM((D, WIN), jnp.float32),
            pltpu.VMEM((D, ENT - TAIL_E), jnp.float32),
            pltpu.VMEM((16, 2 * D), jnp.float32),
            pltpu.VMEM((16, 2 * D), jnp.float32),
            pltpu.VMEM((16, 2 * D), jnp.float32),
            pltpu.VMEM((16, 2 * D), jnp.float32),
            pltpu.VMEM((16,), jnp.int32),
            pltpu.VMEM((16,), jnp.int32),
            pltpu.VMEM((16,), jnp.int32),
            pltpu.VMEM((16,), jnp.int32),
            pltpu.SemaphoreType.DMA,
            pltpu.SemaphoreType.DMA,
            pltpu.SemaphoreType.DMA,
            pltpu.SemaphoreType.DMA,
            pltpu.SemaphoreType.DMA,
            pltpu.SemaphoreType.DMA,
        ],
        compiler_params=pltpu.CompilerParams(needs_layout_passes=False),
    )
    def k1(ph, pt, nh, nt, tab, tail, out, ibuf, binv, slv, sb, sb2, sbt,
           st0, st1, st2, st3, pid0, pid1, pid2, pid3,
           sem_s, sem_s2, sem0, sem1, sem2, sem3):
        wid = lax.axis_index("s") * num_cores + lax.axis_index("c")
        wb = wid * base_cnt + jnp.minimum(wid, extra)
        wcnt = base_cnt + jnp.where(wid < extra, 1, 0)
        iota = lax.iota(jnp.int32, 16)
        stags = (st0, st1, st2, st3)
        pids = (pid0, pid1, pid2, pid3)
        sems = (sem0, sem1, sem2, sem3)

        for c in range(BIN_CAP // 16):
            binv[pl.ds(c * 16, 16)] = jnp.full((16,), BIG, jnp.int32)
        lo_w, hi_w = wb, wb + wcnt

        def scan_role(role_ref, off):
            def blk(b, off):
                pltpu.sync_copy(role_ref.at[pl.ds(b * 32, 32)], ibuf)

                def row(r, off):
                    def chunk(c, off):
                        v = ibuf[r, pl.ds(c * 16, 16)]
                        widx = v // (WIN // 2)
                        m = (widx >= lo_w) & (widx < hi_w)
                        n = plsc.all_reduce_population_count(m)[0]
                        plsc.store_compressed(
                            binv.at[pl.ds(off, 16)], v, mask=m
                        )
                        return off + n

                    return lax.fori_loop(0, 8, chunk, off)

                return lax.fori_loop(0, 32, row, off)

            return lax.fori_loop(0, n_rows // 32, blk, off)

        off = jnp.int32(0)
        for role in (ph, pt, nh, nt):
            off = scan_role(role, off)
        n_bin_chunks = lax.shift_right_logical(off + 15, 4)

        prime_pair = wb * (WIN // 2)
        for u in range(2):
            pids[u][...] = jnp.broadcast_to(prime_pair, (16,))
            pltpu.async_copy(stags[u], out.at[pids[u]], sems[u])

        def do_window(first_pair, win_e, buf, dma=False):
            fill = jnp.broadcast_to(first_pair, (16,))
            for c in range(SL_CAP // 16):
                slv[pl.ds(c * 16, 16)] = fill

            def p1(c, soff):
                v = binv[pl.ds(c * 16, 16)]
                m = (v >= first_pair) & (v < first_pair + WIN // 2)
                n = plsc.all_reduce_population_count(m)[0]
                plsc.store_compressed(slv.at[pl.ds(soff, 16)], v, mask=m)
                return soff + n

            soff = lax.fori_loop(0, n_bin_chunks, p1, jnp.int32(0))
            n_sl = lax.shift_right_logical(soff + 15, 4)

            def extract4(q, carry):
                for u in range(2):
                    cc = q * 2 + u
                    stag, pid, sem = stags[u], pids[u], sems[u]

                    @pl.when(cc < n_sl)
                    def _():
                        vec = slv[pl.ds(cc * 16, 16)]
                        pltpu.make_async_copy(
                            stag, out.at[pid], sem
                        ).wait()
                        for l in range(16):
                            p = vec[l]
                            e0 = p * 2 - win_e
                            for half in range(2):
                                col = e0 + half
                                cols = jnp.broadcast_to(col, (16,))
                                for k in range(D // 16):
                                    g = plsc.load_gather(
                                        buf, [iota + k * 16, cols]
                                    )
                                    stag[l, pl.ds(half * D + k * 16, 16)] = g
                        pid[...] = vec
                        pltpu.async_copy(stag, out.at[pid], sem)

                return carry

            lax.fori_loop(0, lax.shift_right_logical(n_sl + 1, 1),
                          extract4, 0)

        def stream_fire(i, buf, sem):
            win_e = (wb + i) * WIN
            return pltpu.async_copy(
                tab.at[pl.ds(0, D), pl.ds(win_e, WIN)], buf, sem
            )

        def stream_wait(buf, sem):
            pltpu.make_async_copy(
                tab.at[pl.ds(0, D), pl.ds(0, WIN)], buf, sem
            ).wait()

        def win_pair_body(j, carry):
            for u, (buf, sem) in enumerate(((sb, sem_s), (sb2, sem_s2))):
                i = j * 2 + u

                @pl.when(i < n_full_here)
                def _():
                    @pl.when(i + 1 < n_full_here)
                    def _():
                        nxt = i + 1
                        nbuf, nsem = ((sb2, sem_s2), (sb, sem_s))[u]
                        stream_fire(nxt, nbuf, nsem)

                    stream_wait(buf, sem)
                    win = wb + i
                    do_window(win * (WIN // 2), win * WIN, buf, dma=False)

            return carry

        n_full_here = wcnt - jnp.where(hi_w > NFULL, 1, 0)

        @pl.when(n_full_here > 0)
        def _():
            stream_fire(jnp.int32(0), sb, sem_s)

        lax.fori_loop(0, lax.shift_right_logical(n_full_here + 1, 1),
                      win_pair_body, 0)

        @pl.when(hi_w > NFULL)
        def _():
            pltpu.sync_copy(tail, sbt)
            do_window(jnp.int32(TAIL_E // 2), jnp.int32(TAIL_E), sbt,
                      dma=False)

        for u in range(2):
            pltpu.make_async_copy(
                stags[u], out.at[pids[u]], sems[u]
            ).wait()

    return k1(*pair_idx_2d, ent_t, ent_t_tail)


def _sc_loss(idx2d, pairs, rel2, num_cores, num_subcores, rows_per_worker):
    n_chunks = rows_per_worker // CHUNK
    n_groups = CHUNK // 16
    mesh = plsc.VectorSubcoreMesh(core_axis_name="c", subcore_axis_name="s")
    nw = num_cores * num_subcores

    idx_t = [pltpu.VMEM((n_chunks, CHUNK), jnp.int32) for _ in range(12)]
    buf_t = [pltpu.VMEM((CHUNK, 2 * D), jnp.float32) for _ in range(6)]

    @functools.partial(
        pl.kernel,
        out_type=jax.ShapeDtypeStruct((nw, 16), jnp.float32),
        mesh=mesh,
        scratch_types=idx_t + buf_t + [
            pltpu.VMEM((rows_per_worker,), jnp.float32),
            pltpu.VMEM((16,), jnp.float32),
            pltpu.SemaphoreType.DMA,
            pltpu.SemaphoreType.DMA,
        ],
        compiler_params=pltpu.CompilerParams(needs_layout_passes=False),
    )
    def k2(ph_h, pr_h, pt_h, nh_h, nr_h, nt_h,
           ph_o, pr_o, pt_o, nh_o, nr_o, nt_o, ent, rel, out_hbm,
           phv, prv, ptv, nhv, nrv, ntv,
           pov, prov, ptov, nov, nrov, ntov,
           ha, ra, ta, hb, rb, tb,
           norms, accv, sem_a, sem_b):
        wid = lax.axis_index("s") * num_cores + lax.axis_index("c")
        rbase = wid * n_chunks
        iota = lax.iota(jnp.int32, 16)

        for src, dst in zip(
            (ph_h, pr_h, pt_h, nh_h, nr_h, nt_h,
             ph_o, pr_o, pt_o, nh_o, nr_o, nt_o),
            (phv, prv, ptv, nhv, nrv, ntv,
             pov, prov, ptov, nov, nrov, ntov),
        ):
            pltpu.sync_copy(src.at[pl.ds(rbase, n_chunks)], dst)

        side_idx = ((phv, prv, ptv), (nhv, nrv, ntv))
        side_off = ((pov, prov, ptov), (nov, nrov, ntov))
        buf_sets = ((ha, ra, ta), (hb, rb, tb))
        sems = (sem_a, sem_b)

        def fire(phase):
            side, c = divmod(phase, n_chunks)
            hbuf, rbuf, tbuf = buf_sets[phase % 2]
            sem = sems[phase % 2]
            hi, ri, ti = side_idx[side]
            return [
                pltpu.async_copy(ent.at[hi.at[c]], hbuf, sem),
                pltpu.async_copy(rel.at[ri.at[c]], rbuf, sem),
                pltpu.async_copy(ent.at[ti.at[c]], tbuf, sem),
            ]

        def chunk_norms(phase):
            side, c = divmod(phase, n_chunks)
            hbuf, rbuf, tbuf = buf_sets[phase % 2]
            ho, ro, to = side_off[side]

            def group(g, carry):
                acc_c, _ = carry
                hp16 = ho[c, pl.ds(g * 16, 16)]
                rp16 = ro[c, pl.ds(g * 16, 16)]
                tp16 = to[c, pl.ds(g * 16, 16)]
                nvec = jnp.zeros((16,), jnp.float32)
                for r in range(16):
                    row = g * 16 + r
                    hof, rof, tof = hp16[r], rp16[r], tp16[r]
                    acc = jnp.zeros((16,), jnp.float32)
                    for k in range(D // 16):
                        a = hbuf[row, pl.ds(hof + k * 16, 16)]
                        b = rbuf[row, pl.ds(rof + k * 16, 16)]
                        t = tbuf[row, pl.ds(tof + k * 16, 16)]
                        dv = a + b - t
                        acc = acc + dv * dv
                    nvec = jnp.where(iota == r, jnp.sum(acc), nvec)
                n16 = _vec_sqrt(nvec)
                base = (c * n_groups + g) * 16
                if side == 0:
                    norms[pl.ds(base, 16)] = n16
                    return (acc_c, 0)
                pn = norms[pl.ds(base, 16)]
                return (acc_c + jnp.maximum(GAMMA + pn - n16, 0.0), 0)

            out, _ = lax.fori_loop(
                0, n_groups, group, (jnp.zeros((16,), jnp.float32), 0)
            )
            return out

        n_phases = 2 * n_chunks
        total = jnp.zeros((16,), jnp.float32)
        inflight = {0: fire(0)}
        for ph in range(n_phases):
            if ph + 1 < n_phases:
                inflight[ph + 1] = fire(ph + 1)
            for de in inflight.pop(ph):
                de.wait()
            part = chunk_norms(ph)
            if ph >= n_chunks:
                total = total + part

        accv[...] = total
        pltpu.sync_copy(accv, out_hbm.at[wid])

    return k2(*idx2d, pairs, rel2)


def _tc_reduce(partials):
    def body(x_ref, o_ref):
        o_ref[...] = jnp.sum(x_ref[...]).reshape(1, 1)

    return pl.pallas_call(
        body, out_shape=jax.ShapeDtypeStruct((1, 1), jnp.float32)
    )(partials)


def kernel(pos_head, pos_relation, pos_tail, neg_head, neg_relation,
           neg_tail, entity_embedding, relation_embedding):
    info = plsc.get_sparse_core_info()
    nw = info.num_cores * info.num_subcores
    batch = pos_head.shape[0]
    rows_per_worker = batch // nw
    rel2 = relation_embedding.reshape(-1, 2 * D)
    idx = [a.astype(jnp.int32) for a in
           (pos_head, pos_relation, pos_tail,
            neg_head, neg_relation, neg_tail)]
    halves = [(a >> 1).reshape(batch // CHUNK, CHUNK) for a in idx]
    offs = [((a & 1) * D).reshape(batch // CHUNK, CHUNK) for a in idx]
    pairs = _sc_extract(
        entity_embedding.T,
        entity_embedding[TAIL_E:].T,
        [halves[0], halves[2], halves[3], halves[5]],
        info.num_cores, info.num_subcores,
    )
    partials = _sc_loss(halves + offs, pairs, rel2,
                        info.num_cores, info.num_subcores,
                        rows_per_worker)
    return _tc_reduce(partials)[0, 0]


# speedup vs baseline: 1.2625x; 1.2625x over previous
import functools

import jax
import jax.numpy as jnp
from jax import lax
from jax.experimental import pallas as pl
from jax.experimental.pallas import tpu as pltpu
from jax.experimental.pallas import tpu_sc as plsc

GAMMA = 1.0
CHUNK = 128
D = 64
ENT = 1000000
NPAIR = ENT // 2
WIN = 512
NFULL = ENT // WIN
TAIL_E = NFULL * WIN
BIN_CAP = 3264
SL_CAP = 160
BIG = 0x7FFFFFFF


def _vec_sqrt(x):
    i = lax.bitcast_convert_type(x, jnp.int32)
    y = lax.bitcast_convert_type(
        jnp.int32(0x5F3759DF) - lax.shift_right_arithmetic(i, 1), jnp.float32
    )
    for _ in range(3):
        y = y * (1.5 - 0.5 * x * y * y)
    return x * y


def _sc_extract(ent_t, ent_t_tail, pair_idx_2d, num_cores, num_subcores):
    mesh = plsc.VectorSubcoreMesh(core_axis_name="c", subcore_axis_name="s")
    nw = num_cores * num_subcores
    n_rows = pair_idx_2d[0].shape[0]
    n_win = NFULL + 1
    base_cnt, extra = divmod(n_win, nw)

    @functools.partial(
        pl.kernel,
        out_type=jax.ShapeDtypeStruct((NPAIR, 2 * D), jnp.float32),
        mesh=mesh,
        scratch_types=[
            pltpu.VMEM((32, 128), jnp.int32),
            pltpu.VMEM((BIN_CAP,), jnp.int32),
            pltpu.VMEM((SL_CAP,), jnp.int32),
            pltpu.VMEM((D, WIN), jnp.float32),
            pltpu.VMEM((D, WIN), jnp.float32),
            pltpu.VMEM((D, ENT - TAIL_E), jnp.float32),
            pltpu.VMEM((16, 2 * D), jnp.float32),
            pltpu.VMEM((16, 2 * D), jnp.float32),
            pltpu.VMEM((16, 2 * D), jnp.float32),
            pltpu.VMEM((16, 2 * D), jnp.float32),
            pltpu.VMEM((16,), jnp.int32),
            pltpu.VMEM((16,), jnp.int32),
            pltpu.VMEM((16,), jnp.int32),
            pltpu.VMEM((16,), jnp.int32),
            pltpu.SemaphoreType.DMA,
            pltpu.SemaphoreType.DMA,
            pltpu.SemaphoreType.DMA,
            pltpu.SemaphoreType.DMA,
            pltpu.SemaphoreType.DMA,
            pltpu.SemaphoreType.DMA,
        ],
        compiler_params=pltpu.CompilerParams(needs_layout_passes=False),
    )
    def k1(ph, pt, nh, nt, tab, tail, out, ibuf, binv, slv, sb, sb2, sbt,
           st0, st1, st2, st3, pid0, pid1, pid2, pid3,
           sem_s, sem_s2, sem0, sem1, sem2, sem3):
        wid = lax.axis_index("s") * num_cores + lax.axis_index("c")
        wb = wid * base_cnt + jnp.minimum(wid, extra)
        wcnt = base_cnt + jnp.where(wid < extra, 1, 0)
        iota = lax.iota(jnp.int32, 16)
        stags = (st0, st1, st2, st3)
        pids = (pid0, pid1, pid2, pid3)
        sems = (sem0, sem1, sem2, sem3)

        for c in range(BIN_CAP // 16):
            binv[pl.ds(c * 16, 16)] = jnp.full((16,), BIG, jnp.int32)
        lo_w, hi_w = wb, wb + wcnt

        def scan_role(role_ref, off):
            def blk(b, off):
                pltpu.sync_copy(role_ref.at[pl.ds(b * 32, 32)], ibuf)

                def row(r, off):
                    def chunk(c, off):
                        v = ibuf[r, pl.ds(c * 16, 16)]
                        widx = lax.shift_right_logical(v, 8)
                        m = (widx >= lo_w) & (widx < hi_w)
                        n = plsc.all_reduce_population_count(m)[0]
                        plsc.store_compressed(
                            binv.at[pl.ds(off, 16)], v, mask=m
                        )
                        return off + n

                    return lax.fori_loop(0, 8, chunk, off)

                return lax.fori_loop(0, 32, row, off)

            return lax.fori_loop(0, n_rows // 32, blk, off)

        off = jnp.int32(0)
        for role in (ph, pt, nh, nt):
            off = scan_role(role, off)
        n_bin_chunks = lax.shift_right_logical(off + 15, 4)

        prime_pair = wb * (WIN // 2)
        for u in range(2):
            pids[u][...] = jnp.broadcast_to(prime_pair, (16,))
            pltpu.async_copy(stags[u], out.at[pids[u]], sems[u])

        def do_window(first_pair, win_e, buf, dma=False):
            fill = jnp.broadcast_to(first_pair, (16,))
            for c in range(SL_CAP // 16):
                slv[pl.ds(c * 16, 16)] = fill

            def p1(c, soff):
                v = binv[pl.ds(c * 16, 16)]
                m = (v >= first_pair) & (v < first_pair + WIN // 2)
                n = plsc.all_reduce_population_count(m)[0]
                plsc.store_compressed(slv.at[pl.ds(soff, 16)], v, mask=m)
                return soff + n

            soff = lax.fori_loop(0, n_bin_chunks, p1, jnp.int32(0))
            n_sl = lax.shift_right_logical(soff + 15, 4)

            def extract4(q, carry):
                for u in range(2):
                    cc = q * 2 + u
                    stag, pid, sem = stags[u], pids[u], sems[u]

                    @pl.when(cc < n_sl)
                    def _():
                        vec = slv[pl.ds(cc * 16, 16)]
                        pltpu.make_async_copy(
                            stag, out.at[pid], sem
                        ).wait()
                        for l in range(16):
                            p = vec[l]
                            e0 = p * 2 - win_e
                            for half in range(2):
                                col = e0 + half
                                cols = jnp.broadcast_to(col, (16,))
                                for k in range(D // 16):
                                    g = plsc.load_gather(
                                        buf, [iota + k * 16, cols]
                                    )
                                    stag[l, pl.ds(half * D + k * 16, 16)] = g
                        pid[...] = vec
                        pltpu.async_copy(stag, out.at[pid], sem)

                return carry

            lax.fori_loop(0, lax.shift_right_logical(n_sl + 1, 1),
                          extract4, 0)

        def stream_fire(i, buf, sem):
            win_e = (wb + i) * WIN
            return pltpu.async_copy(
                tab.at[pl.ds(0, D), pl.ds(win_e, WIN)], buf, sem
            )

        def stream_wait(buf, sem):
            pltpu.make_async_copy(
                tab.at[pl.ds(0, D), pl.ds(0, WIN)], buf, sem
            ).wait()

        def win_pair_body(j, carry):
            for u, (buf, sem) in enumerate(((sb, sem_s), (sb2, sem_s2))):
                i = j * 2 + u

                @pl.when(i < n_full_here)
                def _():
                    @pl.when(i + 1 < n_full_here)
                    def _():
                        nxt = i + 1
                        nbuf, nsem = ((sb2, sem_s2), (sb, sem_s))[u]
                        stream_fire(nxt, nbuf, nsem)

                    stream_wait(buf, sem)
                    win = wb + i
                    do_window(win * (WIN // 2), win * WIN, buf, dma=False)

            return carry

        n_full_here = wcnt - jnp.where(hi_w > NFULL, 1, 0)

        @pl.when(n_full_here > 0)
        def _():
            stream_fire(jnp.int32(0), sb, sem_s)

        lax.fori_loop(0, lax.shift_right_logical(n_full_here + 1, 1),
                      win_pair_body, 0)

        @pl.when(hi_w > NFULL)
        def _():
            pltpu.sync_copy(tail, sbt)
            do_window(jnp.int32(TAIL_E // 2), jnp.int32(TAIL_E), sbt,
                      dma=False)

        for u in range(2):
            pltpu.make_async_copy(
                stags[u], out.at[pids[u]], sems[u]
            ).wait()

    return k1(*pair_idx_2d, ent_t, ent_t_tail)


def _sc_loss(idx2d, pairs, rel2, num_cores, num_subcores, rows_per_worker):
    n_chunks = rows_per_worker // CHUNK
    n_groups = CHUNK // 16
    mesh = plsc.VectorSubcoreMesh(core_axis_name="c", subcore_axis_name="s")
    nw = num_cores * num_subcores

    idx_t = [pltpu.VMEM((n_chunks, CHUNK), jnp.int32) for _ in range(12)]
    buf_t = [pltpu.VMEM((CHUNK, 2 * D), jnp.float32) for _ in range(6)]

    @functools.partial(
        pl.kernel,
        out_type=jax.ShapeDtypeStruct((nw, 16), jnp.float32),
        mesh=mesh,
        scratch_types=idx_t + buf_t + [
            pltpu.VMEM((rows_per_worker,), jnp.float32),
            pltpu.VMEM((16,), jnp.float32),
            pltpu.SemaphoreType.DMA,
            pltpu.SemaphoreType.DMA,
        ],
        compiler_params=pltpu.CompilerParams(needs_layout_passes=False),
    )
    def k2(ph_h, pr_h, pt_h, nh_h, nr_h, nt_h,
           ph_o, pr_o, pt_o, nh_o, nr_o, nt_o, ent, rel, out_hbm,
           phv, prv, ptv, nhv, nrv, ntv,
           pov, prov, ptov, nov, nrov, ntov,
           ha, ra, ta, hb, rb, tb,
           norms, accv, sem_a, sem_b):
        wid = lax.axis_index("s") * num_cores + lax.axis_index("c")
        rbase = wid * n_chunks
        iota = lax.iota(jnp.int32, 16)

        for src, dst in zip(
            (ph_h, pr_h, pt_h, nh_h, nr_h, nt_h,
             ph_o, pr_o, pt_o, nh_o, nr_o, nt_o),
            (phv, prv, ptv, nhv, nrv, ntv,
             pov, prov, ptov, nov, nrov, ntov),
        ):
            pltpu.sync_copy(src.at[pl.ds(rbase, n_chunks)], dst)

        side_idx = ((phv, prv, ptv), (nhv, nrv, ntv))
        side_off = ((pov, prov, ptov), (nov, nrov, ntov))
        buf_sets = ((ha, ra, ta), (hb, rb, tb))
        sems = (sem_a, sem_b)

        def fire(phase):
            side, c = divmod(phase, n_chunks)
            hbuf, rbuf, tbuf = buf_sets[phase % 2]
            sem = sems[phase % 2]
            hi, ri, ti = side_idx[side]
            return [
                pltpu.async_copy(ent.at[hi.at[c]], hbuf, sem),
                pltpu.async_copy(rel.at[ri.at[c]], rbuf, sem),
                pltpu.async_copy(ent.at[ti.at[c]], tbuf, sem),
            ]

        def chunk_norms(phase):
            side, c = divmod(phase, n_chunks)
            hbuf, rbuf, tbuf = buf_sets[phase % 2]
            ho, ro, to = side_off[side]

            def group(g, carry):
                acc_c, _ = carry
                hp16 = ho[c, pl.ds(g * 16, 16)]
                rp16 = ro[c, pl.ds(g * 16, 16)]
                tp16 = to[c, pl.ds(g * 16, 16)]
                nvec = jnp.zeros((16,), jnp.float32)
                for r in range(16):
                    row = g * 16 + r
                    hof, rof, tof = hp16[r], rp16[r], tp16[r]
                    acc = jnp.zeros((16,), jnp.float32)
                    for k in range(D // 16):
                        a = hbuf[row, pl.ds(hof + k * 16, 16)]
                        b = rbuf[row, pl.ds(rof + k * 16, 16)]
                        t = tbuf[row, pl.ds(tof + k * 16, 16)]
                        dv = a + b - t
                        acc = acc + dv * dv
                    nvec = jnp.where(iota == r, jnp.sum(acc), nvec)
                n16 = _vec_sqrt(nvec)
                base = (c * n_groups + g) * 16
                if side == 0:
                    norms[pl.ds(base, 16)] = n16
                    return (acc_c, 0)
                pn = norms[pl.ds(base, 16)]
                return (acc_c + jnp.maximum(GAMMA + pn - n16, 0.0), 0)

            out, _ = lax.fori_loop(
                0, n_groups, group, (jnp.zeros((16,), jnp.float32), 0)
            )
            return out

        n_phases = 2 * n_chunks
        total = jnp.zeros((16,), jnp.float32)
        inflight = {0: fire(0)}
        for ph in range(n_phases):
            if ph + 1 < n_phases:
                inflight[ph + 1] = fire(ph + 1)
            for de in inflight.pop(ph):
                de.wait()
            part = chunk_norms(ph)
            if ph >= n_chunks:
                total = total + part

        accv[...] = total
        pltpu.sync_copy(accv, out_hbm.at[wid])

    return k2(*idx2d, pairs, rel2)


def _tc_reduce(partials):
    def body(x_ref, o_ref):
        o_ref[...] = jnp.sum(x_ref[...]).reshape(1, 1)

    return pl.pallas_call(
        body, out_shape=jax.ShapeDtypeStruct((1, 1), jnp.float32)
    )(partials)


def kernel(pos_head, pos_relation, pos_tail, neg_head, neg_relation,
           neg_tail, entity_embedding, relation_embedding):
    info = plsc.get_sparse_core_info()
    nw = info.num_cores * info.num_subcores
    batch = pos_head.shape[0]
    rows_per_worker = batch // nw
    rel2 = relation_embedding.reshape(-1, 2 * D)
    idx = [a.astype(jnp.int32) for a in
           (pos_head, pos_relation, pos_tail,
            neg_head, neg_relation, neg_tail)]
    halves = [(a >> 1).reshape(batch // CHUNK, CHUNK) for a in idx]
    offs = [((a & 1) * D).reshape(batch // CHUNK, CHUNK) for a in idx]
    pairs = _sc_extract(
        entity_embedding.T,
        entity_embedding[TAIL_E:].T,
        [halves[0], halves[2], halves[3], halves[5]],
        info.num_cores, info.num_subcores,
    )
    partials = _sc_loss(halves + offs, pairs, rel2,
                        info.num_cores, info.num_subcores,
                        rows_per_worker)
    return _tc_reduce(partials)[0, 0]


# speedup vs baseline: 1.4391x; 1.1399x over previous
import functools

import jax
import jax.numpy as jnp
from jax import lax
from jax.experimental import pallas as pl
from jax.experimental.pallas import tpu as pltpu
from jax.experimental.pallas import tpu_sc as plsc

GAMMA = 1.0
CHUNK = 128
D = 64
ENT = 1000000
NPAIR = ENT // 2
WIN = 512
NFULL = ENT // WIN
TAIL_E = NFULL * WIN
BIN_CAP = 3264
SL_CAP = 160
BIG = 0x7FFFFFFF


def _vec_sqrt(x):
    i = lax.bitcast_convert_type(x, jnp.int32)
    y = lax.bitcast_convert_type(
        jnp.int32(0x5F3759DF) - lax.shift_right_arithmetic(i, 1), jnp.float32
    )
    for _ in range(3):
        y = y * (1.5 - 0.5 * x * y * y)
    return x * y


def _sc_extract(ent_t, ent_t_tail, pair_idx_2d, num_cores, num_subcores):
    mesh = plsc.VectorSubcoreMesh(core_axis_name="c", subcore_axis_name="s")
    nw = num_cores * num_subcores
    n_rows = pair_idx_2d[0].shape[0]
    n_win = NFULL + 1
    base_cnt, extra = divmod(n_win, nw)

    @functools.partial(
        pl.kernel,
        out_type=jax.ShapeDtypeStruct((NPAIR, 2 * D), jnp.float32),
        mesh=mesh,
        scratch_types=[
            pltpu.VMEM((32, 128), jnp.int32),
            pltpu.VMEM((BIN_CAP,), jnp.int32),
            pltpu.VMEM((8, 448), jnp.int32),
            pltpu.VMEM((8, 16), jnp.int32),
            pltpu.VMEM((SL_CAP,), jnp.int32),
            pltpu.VMEM((D, WIN), jnp.float32),
            pltpu.VMEM((D, WIN), jnp.float32),
            pltpu.VMEM((D, ENT - TAIL_E), jnp.float32),
            pltpu.VMEM((16, 2 * D), jnp.float32),
            pltpu.VMEM((16, 2 * D), jnp.float32),
            pltpu.VMEM((16, 2 * D), jnp.float32),
            pltpu.VMEM((16, 2 * D), jnp.float32),
            pltpu.VMEM((16,), jnp.int32),
            pltpu.VMEM((16,), jnp.int32),
            pltpu.VMEM((16,), jnp.int32),
            pltpu.VMEM((16,), jnp.int32),
            pltpu.SemaphoreType.DMA,
            pltpu.SemaphoreType.DMA,
            pltpu.SemaphoreType.DMA,
            pltpu.SemaphoreType.DMA,
            pltpu.SemaphoreType.DMA,
            pltpu.SemaphoreType.DMA,
        ],
        compiler_params=pltpu.CompilerParams(needs_layout_passes=False),
    )
    def k1(ph, pt, nh, nt, tab, tail, out, ibuf, binv, binv2, cnts, slv,
           sb, sb2, sbt,
           st0, st1, st2, st3, pid0, pid1, pid2, pid3,
           sem_s, sem_s2, sem0, sem1, sem2, sem3):
        wid = lax.axis_index("s") * num_cores + lax.axis_index("c")
        wb = wid * base_cnt + jnp.minimum(wid, extra)
        wcnt = base_cnt + jnp.where(wid < extra, 1, 0)
        iota = lax.iota(jnp.int32, 16)
        stags = (st0, st1, st2, st3)
        pids = (pid0, pid1, pid2, pid3)
        sems = (sem0, sem1, sem2, sem3)

        for c in range(BIN_CAP // 16):
            binv[pl.ds(c * 16, 16)] = jnp.full((16,), BIG, jnp.int32)
        lo_w, hi_w = wb, wb + wcnt

        def scan_role(role_ref, off):
            def blk(b, off):
                pltpu.sync_copy(role_ref.at[pl.ds(b * 32, 32)], ibuf)

                def row(r, off):
                    def chunk(c, off):
                        v = ibuf[r, pl.ds(c * 16, 16)]
                        widx = lax.shift_right_logical(v, 8)
                        m = (widx >= lo_w) & (widx < hi_w)
                        n = plsc.all_reduce_population_count(m)[0]
                        plsc.store_compressed(
                            binv.at[pl.ds(off, 16)], v, mask=m
                        )
                        return off + n

                    return lax.fori_loop(0, 8, chunk, off)

                return lax.fori_loop(0, 32, row, off)

            return lax.fori_loop(0, n_rows // 32, blk, off)

        off = jnp.int32(0)
        for role in (ph, pt, nh, nt):
            off = scan_role(role, off)
        n_bin_chunks = lax.shift_right_logical(off + 15, 4)

        big16 = jnp.full((16,), BIG, jnp.int32)
        for r in range(8):
            lo_r = wb + r * 8
            hi_r = jnp.minimum(lo_r + 8, wb + wcnt)

            def part(c, offr, lo_r=lo_r, hi_r=hi_r, r=r):
                v = binv[pl.ds(c * 16, 16)]
                w = lax.shift_right_logical(v, 8)
                m = (w >= lo_r) & (w < hi_r)
                n = plsc.all_reduce_population_count(m)[0]
                plsc.store_compressed(
                    binv2.at[r, pl.ds(offr, 16)], v, mask=m
                )
                return offr + n

            offr = lax.fori_loop(0, n_bin_chunks, part, jnp.int32(0))
            binv2[r, pl.ds(offr, 16)] = big16
            cnts[r, pl.ds(0, 16)] = jnp.broadcast_to(offr, (16,))

        prime_pair = wb * (WIN // 2)
        for u in range(2):
            pids[u][...] = jnp.broadcast_to(prime_pair, (16,))
            pltpu.async_copy(stags[u], out.at[pids[u]], sems[u])

        def do_window(first_pair, win_e, buf, dma=False):
            fill = jnp.broadcast_to(first_pair, (16,))
            for c in range(SL_CAP // 16):
                slv[pl.ds(c * 16, 16)] = fill

            sbid = lax.shift_right_logical(
                lax.shift_right_logical(first_pair, 8) - wb, 3
            )
            cnt = cnts[sbid, pl.ds(0, 16)][0]
            nch = lax.shift_right_logical(cnt + 15, 4)

            def p1(c, soff):
                v = binv2[sbid, pl.ds(c * 16, 16)]
                m = (v >= first_pair) & (v < first_pair + WIN // 2)
                n = plsc.all_reduce_population_count(m)[0]
                plsc.store_compressed(slv.at[pl.ds(soff, 16)], v, mask=m)
                return soff + n

            soff = lax.fori_loop(0, nch, p1, jnp.int32(0))
            n_sl = lax.shift_right_logical(soff + 15, 4)

            def extract4(q, carry):
                for u in range(2):
                    cc = q * 2 + u
                    stag, pid, sem = stags[u], pids[u], sems[u]

                    @pl.when(cc < n_sl)
                    def _():
                        vec = slv[pl.ds(cc * 16, 16)]
                        pltpu.make_async_copy(
                            stag, out.at[pid], sem
                        ).wait()
                        for l in range(16):
                            p = vec[l]
                            e0 = p * 2 - win_e
                            for half in range(2):
                                col = e0 + half
                                cols = jnp.broadcast_to(col, (16,))
                                for k in range(D // 16):
                                    g = plsc.load_gather(
                                        buf, [iota + k * 16, cols]
                                    )
                                    stag[l, pl.ds(half * D + k * 16, 16)] = g
                        pid[...] = vec
                        pltpu.async_copy(stag, out.at[pid], sem)

                return carry

            lax.fori_loop(0, lax.shift_right_logical(n_sl + 1, 1),
                          extract4, 0)

        def stream_fire(i, buf, sem):
            win_e = (wb + i) * WIN
            return pltpu.async_copy(
                tab.at[pl.ds(0, D), pl.ds(win_e, WIN)], buf, sem
            )

        def stream_wait(buf, sem):
            pltpu.make_async_copy(
                tab.at[pl.ds(0, D), pl.ds(0, WIN)], buf, sem
            ).wait()

        def win_pair_body(j, carry):
            for u, (buf, sem) in enumerate(((sb, sem_s), (sb2, sem_s2))):
                i = j * 2 + u

                @pl.when(i < n_full_here)
                def _():
                    @pl.when(i + 1 < n_full_here)
                    def _():
                        nxt = i + 1
                        nbuf, nsem = ((sb2, sem_s2), (sb, sem_s))[u]
                        stream_fire(nxt, nbuf, nsem)

                    stream_wait(buf, sem)
                    win = wb + i
                    do_window(win * (WIN // 2), win * WIN, buf, dma=False)

            return carry

        n_full_here = wcnt - jnp.where(hi_w > NFULL, 1, 0)

        @pl.when(n_full_here > 0)
        def _():
            stream_fire(jnp.int32(0), sb, sem_s)

        lax.fori_loop(0, lax.shift_right_logical(n_full_here + 1, 1),
                      win_pair_body, 0)

        @pl.when(hi_w > NFULL)
        def _():
            pltpu.sync_copy(tail, sbt)
            do_window(jnp.int32(TAIL_E // 2), jnp.int32(TAIL_E), sbt,
                      dma=False)

        for u in range(2):
            pltpu.make_async_copy(
                stags[u], out.at[pids[u]], sems[u]
            ).wait()

    return k1(*pair_idx_2d, ent_t, ent_t_tail)


def _sc_loss(idx2d, pairs, rel2, num_cores, num_subcores, rows_per_worker):
    n_chunks = rows_per_worker // CHUNK
    n_groups = CHUNK // 16
    mesh = plsc.VectorSubcoreMesh(core_axis_name="c", subcore_axis_name="s")
    nw = num_cores * num_subcores

    idx_t = [pltpu.VMEM((n_chunks, CHUNK), jnp.int32) for _ in range(12)]
    buf_t = [pltpu.VMEM((CHUNK, 2 * D), jnp.float32) for _ in range(6)]

    @functools.partial(
        pl.kernel,
        out_type=jax.ShapeDtypeStruct((nw, 16), jnp.float32),
        mesh=mesh,
        scratch_types=idx_t + buf_t + [
            pltpu.VMEM((rows_per_worker,), jnp.float32),
            pltpu.VMEM((16,), jnp.float32),
            pltpu.SemaphoreType.DMA,
            pltpu.SemaphoreType.DMA,
        ],
        compiler_params=pltpu.CompilerParams(needs_layout_passes=False),
    )
    def k2(ph_h, pr_h, pt_h, nh_h, nr_h, nt_h,
           ph_o, pr_o, pt_o, nh_o, nr_o, nt_o, ent, rel, out_hbm,
           phv, prv, ptv, nhv, nrv, ntv,
           pov, prov, ptov, nov, nrov, ntov,
           ha, ra, ta, hb, rb, tb,
           norms, accv, sem_a, sem_b):
        wid = lax.axis_index("s") * num_cores + lax.axis_index("c")
        rbase = wid * n_chunks
        iota = lax.iota(jnp.int32, 16)

        for src, dst in zip(
            (ph_h, pr_h, pt_h, nh_h, nr_h, nt_h,
             ph_o, pr_o, pt_o, nh_o, nr_o, nt_o),
            (phv, prv, ptv, nhv, nrv, ntv,
             pov, prov, ptov, nov, nrov, ntov),
        ):
            pltpu.sync_copy(src.at[pl.ds(rbase, n_chunks)], dst)

        side_idx = ((phv, prv, ptv), (nhv, nrv, ntv))
        side_off = ((pov, prov, ptov), (nov, nrov, ntov))
        buf_sets = ((ha, ra, ta), (hb, rb, tb))
        sems = (sem_a, sem_b)

        def fire(phase):
            side, c = divmod(phase, n_chunks)
            hbuf, rbuf, tbuf = buf_sets[phase % 2]
            sem = sems[phase % 2]
            hi, ri, ti = side_idx[side]
            return [
                pltpu.async_copy(ent.at[hi.at[c]], hbuf, sem),
                pltpu.async_copy(rel.at[ri.at[c]], rbuf, sem),
                pltpu.async_copy(ent.at[ti.at[c]], tbuf, sem),
            ]

        def chunk_norms(phase):
            side, c = divmod(phase, n_chunks)
            hbuf, rbuf, tbuf = buf_sets[phase % 2]
            ho, ro, to = side_off[side]

            def group(g, carry):
                acc_c, _ = carry
                hp16 = ho[c, pl.ds(g * 16, 16)]
                rp16 = ro[c, pl.ds(g * 16, 16)]
                tp16 = to[c, pl.ds(g * 16, 16)]
                nvec = jnp.zeros((16,), jnp.float32)
                for r in range(16):
                    row = g * 16 + r
                    hof, rof, tof = hp16[r], rp16[r], tp16[r]
                    acc = jnp.zeros((16,), jnp.float32)
                    for k in range(D // 16):
                        a = hbuf[row, pl.ds(hof + k * 16, 16)]
                        b = rbuf[row, pl.ds(rof + k * 16, 16)]
                        t = tbuf[row, pl.ds(tof + k * 16, 16)]
                        dv = a + b - t
                        acc = acc + dv * dv
                    nvec = jnp.where(iota == r, jnp.sum(acc), nvec)
                n16 = _vec_sqrt(nvec)
                base = (c * n_groups + g) * 16
                if side == 0:
                    norms[pl.ds(base, 16)] = n16
                    return (acc_c, 0)
                pn = norms[pl.ds(base, 16)]
                return (acc_c + jnp.maximum(GAMMA + pn - n16, 0.0), 0)

            out, _ = lax.fori_loop(
                0, n_groups, group, (jnp.zeros((16,), jnp.float32), 0)
            )
            return out

        n_phases = 2 * n_chunks
        total = jnp.zeros((16,), jnp.float32)
        inflight = {0: fire(0)}
        for ph in range(n_phases):
            if ph + 1 < n_phases:
                inflight[ph + 1] = fire(ph + 1)
            for de in inflight.pop(ph):
                de.wait()
            part = chunk_norms(ph)
            if ph >= n_chunks:
                total = total + part

        accv[...] = total
        pltpu.sync_copy(accv, out_hbm.at[wid])

    return k2(*idx2d, pairs, rel2)


def _tc_reduce(partials):
    def body(x_ref, o_ref):
        o_ref[...] = jnp.sum(x_ref[...]).reshape(1, 1)

    return pl.pallas_call(
        body, out_shape=jax.ShapeDtypeStruct((1, 1), jnp.float32)
    )(partials)


def kernel(pos_head, pos_relation, pos_tail, neg_head, neg_relation,
           neg_tail, entity_embedding, relation_embedding):
    info = plsc.get_sparse_core_info()
    nw = info.num_cores * info.num_subcores
    batch = pos_head.shape[0]
    rows_per_worker = batch // nw
    rel2 = relation_embedding.reshape(-1, 2 * D)
    idx = [a.astype(jnp.int32) for a in
           (pos_head, pos_relation, pos_tail,
            neg_head, neg_relation, neg_tail)]
    halves = [(a >> 1).reshape(batch // CHUNK, CHUNK) for a in idx]
    offs = [((a & 1) * D).reshape(batch // CHUNK, CHUNK) for a in idx]
    pairs = _sc_extract(
        entity_embedding.T,
        entity_embedding[TAIL_E:].T,
        [halves[0], halves[2], halves[3], halves[5]],
        info.num_cores, info.num_subcores,
    )
    partials = _sc_loss(halves + offs, pairs, rel2,
                        info.num_cores, info.num_subcores,
                        rows_per_worker)
    return _tc_reduce(partials)[0, 0]
